# baseline (speedup 1.0000x reference)
import functools

import numpy as np
import jax
import jax.numpy as jnp
from jax import lax
from jax.experimental import pallas as pl
from jax.experimental.pallas import tpu as pltpu
from jax.experimental.pallas import tpu_sc as plsc

_LANES = 16


def _positional_encoding_np(d_model, length):
    pos = np.arange(length, dtype=np.float32)[:, None]
    div = np.exp(
        np.arange(0, d_model, 2, dtype=np.float32) * (-np.log(10000.0) / d_model)
    )
    pe = np.zeros((length, d_model), dtype=np.float32)
    pe[:, 0::2] = np.sin(pos * div)
    pe[:, 1::2] = np.cos(pos * div)
    return pe


def kernel(x, table):
    B, L = x.shape
    V, D = table.shape
    N = B * L
    DD = 2 * D

    NC, NS = 2, 16
    NW = NC * NS
    per_w = N // NW
    assert per_w * NW == N

    RPC = 512
    PPC = RPC // 2
    NCH = per_w // RPC
    assert NCH * RPC == per_w
    G = 128
    NG = RPC // G

    xf = x.reshape(N).astype(jnp.int32)
    table2 = table.reshape(V // 2, DD)
    pe2 = jnp.asarray(_positional_encoding_np(D, L).reshape(L // 2, DD))

    mesh = plsc.VectorSubcoreMesh(core_axis_name="c", subcore_axis_name="s")

    @functools.partial(
        pl.kernel,
        mesh=mesh,
        out_type=jax.ShapeDtypeStruct((N // 2, DD), jnp.float32),
        scratch_types=[
            pltpu.VMEM((L // 2, DD), jnp.float32),
            pltpu.VMEM((RPC,), jnp.int32),
            pltpu.VMEM((RPC,), jnp.int32),
            pltpu.VMEM((RPC, DD), jnp.float32),
            pltpu.VMEM((PPC, DD), jnp.float32),
            pltpu.SemaphoreType.DMA,
        ],
    )
    def run(xf_hbm, table_hbm, pe_hbm, out_hbm, pe_v, idx_v, pidx_v, rows_v, outc_v, sem):
        wid = lax.axis_index("s") * NC + lax.axis_index("c")
        base = wid * per_w
        pltpu.sync_copy(pe_hbm, pe_v)

        def chunk_body(g, carry):
            rbase = base + g * RPC
            pltpu.sync_copy(xf_hbm.at[pl.ds(rbase, RPC)], idx_v)

            def shift_body(k, c2):
                sl = pl.ds(k * _LANES, _LANES)
                pidx_v[sl] = lax.shift_right_logical(idx_v[sl], 1)
                return c2

            lax.fori_loop(0, RPC // _LANES, shift_body, 0)

            copies = [
                pltpu.async_copy(
                    table_hbm.at[pidx_v.at[pl.ds(k * G, G)]],
                    rows_v.at[pl.ds(k * G, G)],
                    sem,
                )
                for k in range(NG)
            ]
            for c in copies:
                c.wait()

            ph0 = lax.rem(g * (RPC % L) // 2, L // 2)

            def add_group(gt, ph):
                j0 = gt * _LANES
                offv = (idx_v[pl.ds(j0, _LANES)] & 1) * D
                for k in range(_LANES):
                    p_local = k // 2
                    half = k % 2
                    off = offv[k]
                    php = ph + p_local
                    php = lax.select(php >= L // 2, php - L // 2, php)
                    t = gt * 8 + p_local
                    for q in range(D // _LANES):
                        src = rows_v[j0 + k, pl.ds(off + q * _LANES, _LANES)]
                        pev = pe_v[php, pl.ds(half * D + q * _LANES, _LANES)]
                        outc_v[t, pl.ds(half * D + q * _LANES, _LANES)] = src + pev
                ph = ph + 8
                return lax.select(ph >= L // 2, ph - L // 2, ph)

            lax.fori_loop(0, PPC // 8, add_group, ph0)
            obase = pl.multiple_of(base // 2 + g * PPC, 256)
            pltpu.sync_copy(outc_v, out_hbm.at[pl.ds(obase, PPC)])
            return carry

        lax.fori_loop(0, NCH, chunk_body, 0)

    out = run(xf, table2, pe2)
    return out.reshape(B, L, D)


# speedup vs baseline: 1.3947x; 1.3947x over previous
import functools

import numpy as np
import jax
import jax.numpy as jnp
from jax import lax
from jax.experimental import pallas as pl
from jax.experimental.pallas import tpu as pltpu
from jax.experimental.pallas import tpu_sc as plsc

_LANES = 16


def _positional_encoding_np(d_model, length):
    pos = np.arange(length, dtype=np.float32)[:, None]
    div = np.exp(
        np.arange(0, d_model, 2, dtype=np.float32) * (-np.log(10000.0) / d_model)
    )
    pe = np.zeros((length, d_model), dtype=np.float32)
    pe[:, 0::2] = np.sin(pos * div)
    pe[:, 1::2] = np.cos(pos * div)
    return pe


def kernel(x, table):
    B, L = x.shape
    V, D = table.shape

    NC, NS = 2, 16
    NW = NC * NS

    TB = 256
    NTC = TB // 128
    NBB = B // TB
    NU = (L * NBB) // NW
    assert NU * NW == L * NBB
    TRW = (D // 8) * NTC * 8 * 128
    PIECE = NTC * 8 * 128

    xT = x.astype(jnp.int32).T.reshape(L * B)
    pe = jnp.asarray(_positional_encoding_np(D, L))

    mesh = plsc.VectorSubcoreMesh(core_axis_name="c", subcore_axis_name="s")

    @functools.partial(
        pl.kernel,
        mesh=mesh,
        compiler_params=pltpu.CompilerParams(
            use_tc_tiling_on_sc=False, needs_layout_passes=False
        ),
        out_type=jax.ShapeDtypeStruct((L * D * B,), jnp.float32),
        scratch_types=[
            pltpu.VMEM((L, D), jnp.float32),
            pltpu.VMEM((2, TB), jnp.int32),
            pltpu.VMEM((2 * TB, D), jnp.float32),
            pltpu.VMEM((2 * TRW,), jnp.float32),
            pltpu.SemaphoreType.DMA((2,)),
            pltpu.SemaphoreType.DMA((2,)),
        ],
    )
    def run(xf_hbm, table_hbm, pe_hbm, out_hbm, pe_v, idx_v, rows_v, tr_v, gsem, osem):
        wid = lax.axis_index("s") * NC + lax.axis_index("c")
        pltpu.sync_copy(pe_hbm, pe_v)
        iota = lax.iota(jnp.int32, _LANES)
        consts = []
        for q in range(D // _LANES):
            dvec = iota + q * _LANES
            consts.append((dvec // 8) * PIECE + (dvec % 8) * 128)

        def fetch(t):
            u = wid * NU + t
            l = u // NBB
            bb = u % NBB
            s = t % 2
            pltpu.sync_copy(xf_hbm.at[pl.ds(l * B + bb * TB, TB)], idx_v.at[s])
            for k in range(TB // 128):
                pltpu.async_copy(
                    table_hbm.at[idx_v.at[s, pl.ds(k * 128, 128)]],
                    rows_v.at[pl.ds(s * TB + k * 128, 128), pl.ds(0, D)],
                    gsem.at[s],
                )

        fetch(0)

        def unit_body(t, carry):
            u = wid * NU + t
            l = u // NBB
            bb = u % NBB
            s = t % 2

            @pl.when(t + 1 < NU)
            def _():
                fetch(t + 1)

            for k in range(TB // 128):
                pltpu.make_async_copy(
                    table_hbm.at[idx_v.at[s, pl.ds(0, 128)]],
                    rows_v.at[pl.ds(0, 128), pl.ds(0, D)],
                    gsem.at[s],
                ).wait()

            @pl.when(t >= 2)
            def _():
                for _p in range(D // 8):
                    pltpu.make_async_copy(
                        tr_v.at[pl.ds(0, PIECE)],
                        out_hbm.at[pl.ds(0, PIECE)],
                        osem.at[s],
                    ).wait()

            pe_regs = [pe_v[l, pl.ds(q * _LANES, _LANES)] for q in range(D // _LANES)]

            @plsc.parallel_loop(0, TB, unroll=2)
            def _(j):
                jrow = s * TB + j
                base = s * TRW + (j // 128) * 1024 + (j % 128)
                bvec = jnp.full((_LANES,), base, jnp.int32)
                for q in range(D // _LANES):
                    v = rows_v[jrow, pl.ds(q * _LANES, _LANES)]
                    plsc.store_scatter(tr_v, [consts[q] + bvec], v + pe_regs[q])

            obase = ((l * (D // 8)) * (B // 128) + bb * NTC) * 1024
            for p in range(D // 8):
                pltpu.async_copy(
                    tr_v.at[pl.ds(s * TRW + p * PIECE, PIECE)],
                    out_hbm.at[pl.ds(obase + p * (B // 128) * 1024, PIECE)],
                    osem.at[s],
                )
            return carry

        lax.fori_loop(0, NU, unit_body, 0)

        for s in range(2):
            if NU >= 2 - s:
                for _p in range(D // 8):
                    pltpu.make_async_copy(
                        tr_v.at[pl.ds(0, PIECE)],
                        out_hbm.at[pl.ds(0, PIECE)],
                        osem.at[s],
                    ).wait()

    out_flat = run(xT, table, pe)
    out5 = out_flat.reshape(L, D // 8, B // 128, 8, 128)
    return out5.transpose(2, 4, 0, 1, 3).reshape(B, L, D)
